# Initial kernel scaffold; baseline (speedup 1.0000x reference)
import jax
import jax.numpy as jnp
from jax.experimental import pallas as pl


def kernel(x, edge_index, edge_attr, batch, W1, b1, W2, b2, Wr, br):
    raise NotImplementedError("write your pallas kernel here")


# speedup vs baseline: 1.1510x; 1.1510x over previous
import functools

import jax
import jax.numpy as jnp
from jax import lax
from jax.experimental import pallas as pl
from jax.experimental.pallas import tpu as pltpu
from jax.experimental.pallas import tpu_sc as plsc

N = 10000
E = 160000
D_IN = 41
D_OUT = 32
D_EDGE = 10
EDGE_HID = 32

XP = 48
EA_P = 16
NW = 32
EW = 5120
EP = NW * EW
NCHUNK = EW // 128
GRP = 8
NGRP = NCHUNK // GRP
NPAD = 10240
ROWS_PER_TILE = NPAD // 16


def _sc_gather(table, idx2):
    mesh = plsc.VectorSubcoreMesh(core_axis_name="c", subcore_axis_name="s")

    @functools.partial(
        pl.kernel,
        mesh=mesh,
        out_type=jax.ShapeDtypeStruct((EP, XP), jnp.float32),
        scratch_types=[
            pltpu.VMEM((NCHUNK, 128), jnp.int32),
            pltpu.VMEM((GRP * 128, XP), jnp.float32),
            pltpu.SemaphoreType.DMA,
        ],
        compiler_params=pltpu.CompilerParams(use_tc_tiling_on_sc=False),
    )
    def gk(table_hbm, idx_hbm, out_hbm, idx_v, rows_v, sem):
        cid = lax.axis_index("c")
        sid = lax.axis_index("s")
        wid = sid * 2 + cid
        pltpu.sync_copy(idx_hbm.at[pl.ds(wid * NCHUNK, NCHUNK)], idx_v)

        def grp_body(g, carry):
            cps = []
            for j in range(GRP):
                cps.append(pltpu.async_copy(
                    table_hbm.at[idx_v.at[g * GRP + j]],
                    rows_v.at[pl.ds(j * 128, 128)],
                    sem,
                ))
            for cp in cps:
                cp.wait()
            pltpu.sync_copy(
                rows_v,
                out_hbm.at[pl.ds(wid * EW + g * (GRP * 128), GRP * 128)],
            )
            return carry

        lax.fori_loop(0, NGRP, grp_body, 0)

    return gk(table, idx2)


def _sc_scatter(msg, idx2, zeros_tile):
    mesh = plsc.VectorSubcoreMesh(core_axis_name="c", subcore_axis_name="s")

    @functools.partial(
        pl.kernel,
        mesh=mesh,
        out_type=jax.ShapeDtypeStruct((2, NPAD, XP), jnp.float32),
        scratch_types=[
            pltpu.VMEM((NCHUNK, 128), jnp.int32),
            pltpu.VMEM((GRP * 128, XP), jnp.float32),
            pltpu.VMEM_SHARED((NPAD, XP), jnp.float32),
            pltpu.SemaphoreType.DMA,
        ],
        compiler_params=pltpu.CompilerParams(use_tc_tiling_on_sc=False),
    )
    def sk(zeros_hbm, msg_hbm, idx_hbm, out_hbm, idx_v, rows_v, acc_sh, sem):
        cid = lax.axis_index("c")
        sid = lax.axis_index("s")
        wid = sid * 2 + cid
        pltpu.sync_copy(
            zeros_hbm, acc_sh.at[pl.ds(sid * ROWS_PER_TILE, ROWS_PER_TILE)])
        plsc.subcore_barrier()
        pltpu.sync_copy(idx_hbm.at[pl.ds(wid * NCHUNK, NCHUNK)], idx_v)

        def grp_body(g, carry):
            pltpu.sync_copy(
                msg_hbm.at[pl.ds(wid * EW + g * (GRP * 128), GRP * 128)],
                rows_v)
            for j in range(GRP):
                pltpu.sync_copy(
                    rows_v.at[pl.ds(j * 128, 128)],
                    acc_sh.at[idx_v.at[g * GRP + j]],
                    add=True,
                )
            return carry

        lax.fori_loop(0, NGRP, grp_body, 0)
        plsc.subcore_barrier()
        pltpu.sync_copy(
            acc_sh.at[pl.ds(sid * ROWS_PER_TILE, ROWS_PER_TILE)],
            out_hbm.at[cid, pl.ds(sid * ROWS_PER_TILE, ROWS_PER_TILE)],
        )

    return sk(zeros_tile, msg, idx2)


EB = 2048


def _tc_edge_body(ea_ref, xs_ref, w1_ref, b1_ref, w2_ref, b2_ref, out_ref):
    ea = ea_ref[...]
    h = jnp.maximum(
        jnp.dot(ea, w1_ref[...], preferred_element_type=jnp.float32)
        + b1_ref[...], 0.0)
    T = jnp.dot(h, w2_ref[...], preferred_element_type=jnp.float32) + b2_ref[...]
    xs = xs_ref[...]
    msg = xs[:, 0:1] * T[:, 0:D_OUT]
    for i in range(1, D_IN):
        msg = msg + xs[:, i:i + 1] * T[:, i * D_OUT:(i + 1) * D_OUT]
    out_ref[...] = jnp.concatenate(
        [msg, jnp.ones((EB, XP - D_OUT), jnp.float32)], axis=1)


def _tc_edge(eap, xs, W1p, b1, W2, b2):
    grid = (EP // EB,)
    return pl.pallas_call(
        _tc_edge_body,
        grid=grid,
        in_specs=[
            pl.BlockSpec((EB, EA_P), lambda i: (i, 0)),
            pl.BlockSpec((EB, XP), lambda i: (i, 0)),
            pl.BlockSpec((EA_P, EDGE_HID), lambda i: (0, 0)),
            pl.BlockSpec((1, EDGE_HID), lambda i: (0, 0)),
            pl.BlockSpec((EDGE_HID, D_IN * D_OUT), lambda i: (0, 0)),
            pl.BlockSpec((1, D_IN * D_OUT), lambda i: (0, 0)),
        ],
        out_specs=pl.BlockSpec((EB, XP), lambda i: (i, 0)),
        out_shape=jax.ShapeDtypeStruct((EP, XP), jnp.float32),
    )(eap, xs, W1p, b1.reshape(1, -1), W2, b2.reshape(1, -1))


NB = 1000


def _tc_final_body(x_ref, p_ref, wr_ref, br_ref, out_ref):
    p = p_ref[...]
    s = p[0] + p[1]
    cnt = jnp.maximum(s[:, D_OUT:D_OUT + 1], 1.0)
    agg = s[:, :D_OUT] / cnt
    out = (jnp.dot(x_ref[...], wr_ref[...], preferred_element_type=jnp.float32)
           + br_ref[...] + agg)
    out_ref[...] = jnp.maximum(out, 0.0)


def _tc_final(x1p, parts, Wrp, br):
    grid = (N // NB,)
    return pl.pallas_call(
        _tc_final_body,
        grid=grid,
        in_specs=[
            pl.BlockSpec((NB, XP), lambda i: (i, 0)),
            pl.BlockSpec((2, NB, XP), lambda i: (0, i, 0)),
            pl.BlockSpec((XP, D_OUT), lambda i: (0, 0)),
            pl.BlockSpec((1, D_OUT), lambda i: (0, 0)),
        ],
        out_specs=pl.BlockSpec((NB, D_OUT), lambda i: (i, 0)),
        out_shape=jax.ShapeDtypeStruct((N, D_OUT), jnp.float32),
    )(x1p, parts, Wrp, br.reshape(1, -1))


def kernel(x, edge_index, edge_attr, batch, W1, b1, W2, b2, Wr, br):
    x1p = jnp.pad(x[:, :D_IN], ((0, 0), (0, XP - D_IN)))
    src = edge_index[0]
    dst = edge_index[1]
    pad_e = EP - E
    pad_ar = jnp.arange(pad_e, dtype=jnp.int32)
    src_p = jnp.concatenate([src, pad_ar % N])
    dst_p = jnp.concatenate([dst, N + pad_ar % (NPAD - N)])
    src2 = src_p.reshape(EP // 128, 128)
    dst2 = dst_p.reshape(EP // 128, 128)
    eap = jnp.pad(edge_attr, ((0, pad_e), (0, EA_P - D_EDGE)))
    W1p = jnp.pad(W1, ((0, EA_P - D_EDGE), (0, 0)))
    Wrp = jnp.pad(Wr, ((0, XP - D_IN), (0, 0)))
    zeros_tile = jnp.zeros((ROWS_PER_TILE, XP), jnp.float32)

    xs = _sc_gather(x1p, src2)
    msg = _tc_edge(eap, xs, W1p, b1, W2, b2)
    parts = _sc_scatter(msg, dst2, zeros_tile)
    return _tc_final(x1p, parts, Wrp, br)
